# baseline (speedup 1.0000x reference)
import functools

import jax
import jax.numpy as jnp
from jax import lax
from jax.experimental import pallas as pl
from jax.experimental.pallas import tpu as pltpu
from jax.experimental.pallas import tpu_sc as plsc

_MAX_POSITION = 2048


def _tc_phase_shift(tab_ref, out_ref, *, depth, width):
    k = pl.program_id(0)
    ncols = tab_ref.shape[1]
    rolled = pltpu.roll(tab_ref[...], ncols - k, axis=1)
    out_ref[0] = rolled[:, :width]


def _sc_expand(phases_hbm, out_hbm, sem_out, *, length, rows_per, fire_k):
    c = lax.axis_index("c")
    s = lax.axis_index("s")
    wid = c * 16 + s
    base = wid * rows_per
    for chunk in range(rows_per // fire_k):
        handles = []
        for r in range(fire_k):
            i = base + chunk * fire_k + r
            start = (length - 1) - i
            k = lax.rem(start, 128)
            a = pl.multiple_of(start - k, 128)
            handles.append(
                pltpu.async_copy(
                    phases_hbm.at[k, :, pl.ds(a, length)],
                    out_hbm.at[i],
                    sem_out,
                )
            )
        for h in handles:
            h.wait()


def kernel(inputs, rel_embeddings):
    length = inputs.shape[1]
    depth = rel_embeddings.shape[1]
    table_rows = rel_embeddings.shape[0]

    width = length + 2048 + 128
    padded_cols = width + 128
    revt = rel_embeddings[::-1].T
    revt = jnp.pad(revt, ((0, 0), (0, padded_cols - table_rows)))

    phases = pl.pallas_call(
        functools.partial(_tc_phase_shift, depth=depth, width=width),
        grid=(128,),
        in_specs=[pl.BlockSpec((depth, padded_cols), lambda k: (0, 0))],
        out_specs=pl.BlockSpec((1, depth, width), lambda k: (k, 0, 0)),
        out_shape=jax.ShapeDtypeStruct((128, depth, width), jnp.float32),
    )(revt)

    n_workers = 32
    rows_per = length // n_workers
    mesh = plsc.VectorSubcoreMesh(core_axis_name="c", subcore_axis_name="s")
    body = functools.partial(_sc_expand, length=length, rows_per=rows_per, fire_k=8)
    out = pl.kernel(
        body,
        mesh=mesh,
        out_type=jax.ShapeDtypeStruct((length, depth, length), jnp.float32),
        scratch_types=[pltpu.SemaphoreType.DMA],
    )(phases)
    return jnp.transpose(out, (0, 2, 1))


# speedup vs baseline: 42.0620x; 42.0620x over previous
import functools

import jax
import jax.numpy as jnp
from jax import lax
from jax.experimental import pallas as pl
from jax.experimental.pallas import tpu as pltpu
from jax.experimental.pallas import tpu_sc as plsc

_MAX_POSITION = 2048


def _tc_phase_shift(tab_ref, out_ref, *, width):
    k = pl.program_id(0)
    ncols = tab_ref.shape[1]
    rolled = pltpu.roll(tab_ref[...], ncols - k, axis=1)
    out_ref[0] = rolled[:, :width]


def _sc_expand(phases_hbm, out_hbm, spmem, sem_stage, sem_out, *, length, depth):
    c = lax.axis_index("c")
    s = lax.axis_index("s")
    n_waves = 128 // 2

    def stage(w):
        return pltpu.make_async_copy(
            phases_hbm.at[2 * w + c], spmem.at[lax.rem(w, 2)], sem_stage
        )

    @pl.when(s == 0)
    def _():
        h = stage(0)
        h.start()
        h.wait()

    plsc.subcore_barrier()

    def wave(w, carry):
        nxt = w + 1
        prefetch = (s == 0) & (nxt < n_waves)

        @pl.when(prefetch)
        def _():
            stage(nxt).start()

        phase = 2 * w + c
        i = (128 - 1) - phase + 128 * s
        a = pl.multiple_of(
            (length - 128) - 128 * s, 128
        )
        row = pltpu.make_async_copy(
            spmem.at[lax.rem(w, 2), :, pl.ds(a, length)], out_hbm.at[i], sem_out
        )
        row.start()
        row.wait()

        @pl.when(prefetch)
        def _():
            stage(nxt).wait()

        plsc.subcore_barrier()
        return carry

    lax.fori_loop(0, n_waves, wave, 0)


def kernel(inputs, rel_embeddings):
    length = inputs.shape[1]
    depth = rel_embeddings.shape[1]
    table_rows = rel_embeddings.shape[0]

    width = length + 2048 + 128
    padded_cols = width + 128
    revt = rel_embeddings[::-1].T
    revt = jnp.pad(revt, ((0, 0), (0, padded_cols - table_rows)))

    phases = pl.pallas_call(
        functools.partial(_tc_phase_shift, width=width),
        grid=(128,),
        in_specs=[pl.BlockSpec((depth, padded_cols), lambda k: (0, 0))],
        out_specs=pl.BlockSpec((1, depth, width), lambda k: (k, 0, 0)),
        out_shape=jax.ShapeDtypeStruct((128, depth, width), jnp.float32),
    )(revt)

    mesh = plsc.VectorSubcoreMesh(core_axis_name="c", subcore_axis_name="s")
    body = functools.partial(_sc_expand, length=length, depth=depth)
    out = pl.kernel(
        body,
        mesh=mesh,
        out_type=jax.ShapeDtypeStruct((length, depth, length), jnp.float32),
        scratch_types=[
            pltpu.VMEM_SHARED((2, depth, width), jnp.float32),
            pltpu.SemaphoreType.DMA,
            pltpu.SemaphoreType.DMA,
        ],
    )(phases)
    return jnp.transpose(out, (0, 2, 1))


# speedup vs baseline: 46.1118x; 1.0963x over previous
import functools

import jax
import jax.numpy as jnp
from jax import lax
from jax.experimental import pallas as pl
from jax.experimental.pallas import tpu as pltpu
from jax.experimental.pallas import tpu_sc as plsc

_MAX_POSITION = 2048


def _tc_phase_shift(tab_ref, out_ref, *, width):
    k = pl.program_id(0)
    ncols = tab_ref.shape[1]
    rolled = pltpu.roll(tab_ref[...], ncols - k, axis=1)
    out_ref[0] = rolled[:, :width]


def _sc_expand(phases_hbm, out_hbm, spmem, sem_stage, sem_out, *, length, depth):
    c = lax.axis_index("c")
    s = lax.axis_index("s")
    ppw = 2
    n_waves = 128 // (2 * ppw)

    def stage(w):
        return pltpu.make_async_copy(
            phases_hbm.at[pl.ds(2 * ppw * w + ppw * c, ppw)],
            spmem.at[lax.rem(w, 2)],
            sem_stage,
        )

    @pl.when(s == 0)
    def _():
        h = stage(0)
        h.start()
        h.wait()

    plsc.subcore_barrier()

    def wave(w, carry):
        nxt = w + 1
        prefetch = (s == 0) & (nxt < n_waves)

        @pl.when(prefetch)
        def _():
            stage(nxt).start()

        a = pl.multiple_of(
            (length - 128) - 128 * s, 128
        )
        rows = []
        for q in range(ppw):
            phase = 2 * ppw * w + ppw * c + q
            i = (128 - 1) - phase + 128 * s
            rows.append(
                pltpu.make_async_copy(
                    spmem.at[lax.rem(w, 2), q, :, pl.ds(a, length)],
                    out_hbm.at[i],
                    sem_out,
                )
            )
        for r in rows:
            r.start()
        for r in rows:
            r.wait()

        @pl.when(prefetch)
        def _():
            stage(nxt).wait()

        plsc.subcore_barrier()
        return carry

    lax.fori_loop(0, n_waves, wave, 0)


def kernel(inputs, rel_embeddings):
    length = inputs.shape[1]
    depth = rel_embeddings.shape[1]
    table_rows = rel_embeddings.shape[0]

    width = 2 * length - 128
    padded_cols = width + 128
    revt = rel_embeddings[::-1].T
    revt = jnp.pad(revt, ((0, 0), (0, padded_cols - table_rows)))

    phases = pl.pallas_call(
        functools.partial(_tc_phase_shift, width=width),
        grid=(128,),
        in_specs=[pl.BlockSpec((depth, padded_cols), lambda k: (0, 0))],
        out_specs=pl.BlockSpec((1, depth, width), lambda k: (k, 0, 0)),
        out_shape=jax.ShapeDtypeStruct((128, depth, width), jnp.float32),
    )(revt)

    mesh = plsc.VectorSubcoreMesh(core_axis_name="c", subcore_axis_name="s")
    body = functools.partial(_sc_expand, length=length, depth=depth)
    out = pl.kernel(
        body,
        mesh=mesh,
        out_type=jax.ShapeDtypeStruct((length, depth, length), jnp.float32),
        scratch_types=[
            pltpu.VMEM_SHARED((2, 2, depth, width), jnp.float32),
            pltpu.SemaphoreType.DMA,
            pltpu.SemaphoreType.DMA,
        ],
    )(phases)
    return jnp.transpose(out, (0, 2, 1))


# speedup vs baseline: 48.6694x; 1.0555x over previous
import functools

import jax
import jax.numpy as jnp
from jax import lax
from jax.experimental import pallas as pl
from jax.experimental.pallas import tpu as pltpu
from jax.experimental.pallas import tpu_sc as plsc

_MAX_POSITION = 2048


def _tc_phase_shift(tab_ref, out_ref, *, width):
    ncols = tab_ref.shape[1]
    for t in range(tab_ref.shape[0]):
        x = jnp.broadcast_to(tab_ref[t], (128, ncols))
        rolled = pltpu.roll(x, ncols - 127, axis=1, stride=1, stride_axis=0)
        out_ref[:, t, :] = rolled[:, :width]


def _sc_expand(phases_hbm, out_hbm, spmem, sem_stage, sem_out, *, length, depth):
    c = lax.axis_index("c")
    s = lax.axis_index("s")
    ppw = 2
    n_waves = 128 // (2 * ppw)

    def stage(w):
        return pltpu.make_async_copy(
            phases_hbm.at[pl.ds(2 * ppw * w + ppw * c, ppw)],
            spmem.at[lax.rem(w, 2)],
            sem_stage,
        )

    @pl.when(s == 0)
    def _():
        h = stage(0)
        h.start()
        h.wait()

    plsc.subcore_barrier()

    def wave(w, carry):
        nxt = w + 1
        prefetch = (s == 0) & (nxt < n_waves)

        @pl.when(prefetch)
        def _():
            stage(nxt).start()

        a = pl.multiple_of(
            (length - 128) - 128 * s, 128
        )
        rows = []
        for q in range(ppw):
            r = 2 * ppw * w + ppw * c + q
            i = r + 128 * s
            rows.append(
                pltpu.make_async_copy(
                    spmem.at[lax.rem(w, 2), q, :, pl.ds(a, length)],
                    out_hbm.at[i],
                    sem_out,
                )
            )
        for h in rows:
            h.start()
        for h in rows:
            h.wait()

        @pl.when(prefetch)
        def _():
            stage(nxt).wait()

        plsc.subcore_barrier()
        return carry

    lax.fori_loop(0, n_waves, wave, 0)


def kernel(inputs, rel_embeddings):
    length = inputs.shape[1]
    depth = rel_embeddings.shape[1]
    table_rows = rel_embeddings.shape[0]

    width = 2 * length - 128
    padded_cols = width + 128
    revt = rel_embeddings[::-1].T
    revt = jnp.pad(revt, ((0, 0), (0, padded_cols - table_rows)))

    phases = pl.pallas_call(
        functools.partial(_tc_phase_shift, width=width),
        grid=(depth // 8,),
        in_specs=[pl.BlockSpec((8, padded_cols), lambda d: (d, 0))],
        out_specs=pl.BlockSpec((128, 8, width), lambda d: (0, d, 0)),
        out_shape=jax.ShapeDtypeStruct((128, depth, width), jnp.float32),
    )(revt)

    mesh = plsc.VectorSubcoreMesh(core_axis_name="c", subcore_axis_name="s")
    body = functools.partial(_sc_expand, length=length, depth=depth)
    out = pl.kernel(
        body,
        mesh=mesh,
        out_type=jax.ShapeDtypeStruct((length, depth, length), jnp.float32),
        scratch_types=[
            pltpu.VMEM_SHARED((2, 2, depth, width), jnp.float32),
            pltpu.SemaphoreType.DMA,
            pltpu.SemaphoreType.DMA,
        ],
    )(phases)
    return jnp.transpose(out, (0, 2, 1))
